# baseline (speedup 1.0000x reference)
import functools

import jax
import jax.numpy as jnp
from jax import lax
from jax.experimental import pallas as pl
from jax.experimental.pallas import tpu as pltpu
from jax.experimental.pallas import tpu_sc as plsc


def _build_gather(n_total, hidden, nc, ns):
    nw = nc * ns
    per_w = n_total // nw
    ch = 80
    n_ch = per_w // ch
    nbuf = 8
    ahead = 4

    mesh = plsc.VectorSubcoreMesh(core_axis_name="c", subcore_axis_name="s")

    @functools.partial(
        pl.kernel,
        out_type=jax.ShapeDtypeStruct((n_total, hidden), jnp.float32),
        mesh=mesh,
        scratch_types=[
            pltpu.VMEM((per_w,), jnp.int32),
        ]
        + [pltpu.VMEM((ch, hidden), jnp.float32)] * nbuf
        + [pltpu.SemaphoreType.DMA] * (2 * nbuf),
    )
    def emb(idx_hbm, tbl_hbm, out_hbm, idx_v, *rest):
        rows_v = rest[:nbuf]
        rsem = rest[nbuf : 2 * nbuf]
        wsem = rest[2 * nbuf : 3 * nbuf]
        wid = lax.axis_index("s") * nc + lax.axis_index("c")
        base = wid * per_w

        def idx_slice(g):
            return idx_v.at[pl.ds(g * ch, ch)]

        pltpu.sync_copy(idx_hbm.at[pl.ds(base, per_w)], idx_v)
        for c in range(ahead):
            pltpu.async_copy(tbl_hbm.at[idx_slice(c)], rows_v[c], rsem[c])

        def group(i, carry):
            for b in range(nbuf):
                g = i * nbuf + b
                sa = (b + ahead) % nbuf

                @pl.when(g + ahead < n_ch)
                def _():
                    @pl.when(g >= nbuf - ahead)
                    def _():
                        pltpu.make_async_copy(
                            rows_v[sa],
                            out_hbm.at[pl.ds(base + (g + ahead) * ch, ch)],
                            wsem[sa],
                        ).wait()

                    pltpu.async_copy(
                        tbl_hbm.at[idx_slice(g + ahead)], rows_v[sa], rsem[sa]
                    )

                pltpu.make_async_copy(
                    tbl_hbm.at[idx_slice(g)], rows_v[b], rsem[b]
                ).wait()
                pltpu.async_copy(
                    rows_v[b], out_hbm.at[pl.ds(base + g * ch, ch)], wsem[b]
                )
            return carry

        lax.fori_loop(0, n_ch // nbuf, group, 0)

        for c in range(n_ch - ahead, n_ch):
            b = c % nbuf
            pltpu.make_async_copy(
                rows_v[b], out_hbm.at[pl.ds(base + c * ch, ch)], wsem[b]
            ).wait()

    return emb


def kernel(input_ids, weight):
    b, l = input_ids.shape
    vocab, hidden = weight.shape
    n_total = b * l
    info = plsc.get_sparse_core_info()
    emb = _build_gather(n_total, hidden, info.num_cores, info.num_subcores)
    out = emb(input_ids.reshape(n_total), weight)
    return out.reshape(b, l, hidden)


# speedup vs baseline: 1.0036x; 1.0036x over previous
import functools

import jax
import jax.numpy as jnp
from jax import lax
from jax.experimental import pallas as pl
from jax.experimental.pallas import tpu as pltpu
from jax.experimental.pallas import tpu_sc as plsc


def _build_gather(n_total, hidden, nc, ns):
    nw = nc * ns
    per_w = n_total // nw
    ch = 80
    n_ch = per_w // ch
    nbuf = 8
    ahead = 4

    mesh = plsc.VectorSubcoreMesh(core_axis_name="c", subcore_axis_name="s")

    @functools.partial(
        pl.kernel,
        out_type=jax.ShapeDtypeStruct((n_total, hidden), jnp.float32),
        mesh=mesh,
        scratch_types=[
            pltpu.VMEM((per_w,), jnp.int32),
        ]
        + [pltpu.VMEM((ch, hidden), jnp.float32)] * nbuf
        + [pltpu.SemaphoreType.DMA] * (2 * nbuf + 2),
    )
    def emb(idx_hbm, tbl_hbm, out_hbm, idx_v, *rest):
        rows_v = rest[:nbuf]
        rsem = rest[nbuf : 2 * nbuf]
        wsem = rest[2 * nbuf : 3 * nbuf]
        isem_a, isem_b = rest[3 * nbuf], rest[3 * nbuf + 1]
        wid = lax.axis_index("s") * nc + lax.axis_index("c")
        base = wid * per_w

        def idx_slice(g):
            return idx_v.at[pl.ds(g * ch, ch)]

        head = ahead * ch
        pltpu.async_copy(
            idx_hbm.at[pl.ds(base, head)], idx_v.at[pl.ds(0, head)], isem_a
        )
        pltpu.async_copy(
            idx_hbm.at[pl.ds(base + head, per_w - head)],
            idx_v.at[pl.ds(head, per_w - head)],
            isem_b,
        )
        pltpu.make_async_copy(
            idx_hbm.at[pl.ds(base, head)], idx_v.at[pl.ds(0, head)], isem_a
        ).wait()
        for c in range(ahead):
            pltpu.async_copy(tbl_hbm.at[idx_slice(c)], rows_v[c], rsem[c])
        pltpu.make_async_copy(
            idx_hbm.at[pl.ds(base + head, per_w - head)],
            idx_v.at[pl.ds(head, per_w - head)],
            isem_b,
        ).wait()

        def group(i, carry):
            for b in range(nbuf):
                g = i * nbuf + b
                sa = (b + ahead) % nbuf

                @pl.when(g + ahead < n_ch)
                def _():
                    @pl.when(g >= nbuf - ahead)
                    def _():
                        pltpu.make_async_copy(
                            rows_v[sa],
                            out_hbm.at[pl.ds(base + (g + ahead) * ch, ch)],
                            wsem[sa],
                        ).wait()

                    pltpu.async_copy(
                        tbl_hbm.at[idx_slice(g + ahead)], rows_v[sa], rsem[sa]
                    )

                pltpu.make_async_copy(
                    tbl_hbm.at[idx_slice(g)], rows_v[b], rsem[b]
                ).wait()
                pltpu.async_copy(
                    rows_v[b], out_hbm.at[pl.ds(base + g * ch, ch)], wsem[b]
                )
            return carry

        lax.fori_loop(0, n_ch // nbuf, group, 0)

        for c in range(n_ch - ahead, n_ch):
            b = c % nbuf
            pltpu.make_async_copy(
                rows_v[b], out_hbm.at[pl.ds(base + c * ch, ch)], wsem[b]
            ).wait()

    return emb


def kernel(input_ids, weight):
    b, l = input_ids.shape
    vocab, hidden = weight.shape
    n_total = b * l
    info = plsc.get_sparse_core_info()
    emb = _build_gather(n_total, hidden, info.num_cores, info.num_subcores)
    out = emb(input_ids.reshape(n_total), weight)
    return out.reshape(b, l, hidden)
